# baseline (speedup 1.0000x reference)
import functools

import jax
import jax.numpy as jnp
from jax import lax
from jax.experimental import pallas as pl
from jax.experimental.pallas import tpu as pltpu
from jax.experimental.pallas import tpu_sc as plsc

T = 2048
H = 1024
E = 16
I = 512
TOPK = 2
NG = 4
TG = 2
ISH = 1024
SCALE = 1.0

TBLK = 256
B = 64
NPAD = T * TOPK + E * B
NBLK = NPAD // B
MLEN = 96

NC = 2
NS = 16
NW = NC * NS
TW = T // NW
L = 16


def _shared_router_body(x_ref, sgu_ref, sd_ref, gw_ref,
                        so_ref, e0_ref, e1_ref, w0_ref, w1_ref, cnt_ref,
                        sgu_b, sd_b):
    x = x_ref[...]

    @pl.when(pl.program_id(0) == 0)
    def _():
        sgu_b[...] = sgu_ref[...].astype(jnp.bfloat16)
        sd_b[...] = sd_ref[...].astype(jnp.bfloat16)

    h = jnp.dot(x.astype(jnp.bfloat16), sgu_b[...],
                preferred_element_type=jnp.float32)
    a = h[:, :ISH]
    b = h[:, ISH:]
    g = a * jax.nn.sigmoid(a) * b
    so_ref[...] = jnp.dot(g.astype(jnp.bfloat16), sd_b[...],
                          preferred_element_type=jnp.float32)

    logits = jax.lax.dot_general(
        x, gw_ref[...], (((1,), (1,)), ((), ())),
        preferred_element_type=jnp.float32)
    m = jnp.max(logits, axis=1, keepdims=True)
    ex = jnp.exp(logits - m)
    sc = ex / jnp.sum(ex, axis=1, keepdims=True)

    gs = [jnp.max(sc[:, 4 * k:4 * k + 4], axis=1, keepdims=True)
          for k in range(NG)]
    col = jax.lax.broadcasted_iota(jnp.int32, (TBLK, E), 1)
    colg = col // (E // NG)
    masked = sc
    for j in range(NG):
        r = jnp.zeros((TBLK, 1), jnp.int32)
        for k in range(NG):
            gt = gs[k] > gs[j]
            tie = (gs[k] == gs[j]) & (k < j)
            r = r + jnp.where(gt | tie, 1, 0)
        keep = r < TG
        masked = jnp.where((colg == j) & jnp.logical_not(keep), 0.0, masked)

    m0 = jnp.max(masked, axis=1, keepdims=True)
    i0 = jnp.min(jnp.where(masked == m0, col, E), axis=1, keepdims=True)
    masked2 = jnp.where(col == i0, -1.0, masked)
    m1 = jnp.max(masked2, axis=1, keepdims=True)
    i1 = jnp.min(jnp.where(masked2 == m1, col, E), axis=1, keepdims=True)
    s = m0 + m1 + 1e-20
    e0_ref[...] = i0
    e1_ref[...] = i1
    w0_ref[...] = m0 / s
    w1_ref[...] = m1 / s

    for sub in range(TBLK // TW):
        s0 = i0[sub * TW:(sub + 1) * TW]
        s1 = i1[sub * TW:(sub + 1) * TW]
        cole = jax.lax.broadcasted_iota(jnp.int32, (TW, E), 1)
        cnt = (jnp.sum(jnp.where(s0 == cole, 1, 0), axis=0, keepdims=True)
               + jnp.sum(jnp.where(s1 == cole, 1, 0), axis=0, keepdims=True))
        cnt_ref[sub] = cnt


def _shared_and_router(x, sgu, sd, gw):
    grid = (T // TBLK,)
    return pl.pallas_call(
        _shared_router_body,
        grid=grid,
        in_specs=[
            pl.BlockSpec((TBLK, H), lambda i: (i, 0)),
            pl.BlockSpec((H, 2 * ISH), lambda i: (0, 0)),
            pl.BlockSpec((ISH, H), lambda i: (0, 0)),
            pl.BlockSpec((E, H), lambda i: (0, 0)),
        ],
        out_specs=[
            pl.BlockSpec((TBLK, H), lambda i: (i, 0)),
            pl.BlockSpec((TBLK, 1), lambda i: (i, 0)),
            pl.BlockSpec((TBLK, 1), lambda i: (i, 0)),
            pl.BlockSpec((TBLK, 1), lambda i: (i, 0)),
            pl.BlockSpec((TBLK, 1), lambda i: (i, 0)),
            pl.BlockSpec((TBLK // TW, 1, E), lambda i: (i, 0, 0)),
        ],
        out_shape=[
            jax.ShapeDtypeStruct((T, H), jnp.float32),
            jax.ShapeDtypeStruct((T, 1), jnp.int32),
            jax.ShapeDtypeStruct((T, 1), jnp.int32),
            jax.ShapeDtypeStruct((T, 1), jnp.float32),
            jax.ShapeDtypeStruct((T, 1), jnp.float32),
            jax.ShapeDtypeStruct((NW, 1, E), jnp.int32),
        ],
        scratch_shapes=[
            pltpu.VMEM((H, 2 * ISH), jnp.bfloat16),
            pltpu.VMEM((ISH, H), jnp.bfloat16),
        ],
    )(x, sgu, sd, gw)


_GDN = jax.lax.GatherDimensionNumbers(
    offset_dims=(), collapsed_slice_dims=(0,), start_index_map=(0,))


def _gather16(v, idx):
    return jax.lax.gather(
        v, idx.reshape(L, 1), _GDN, (1,),
        mode=jax.lax.GatherScatterMode.PROMISE_IN_BOUNDS)


def _splat16(v, i):
    return _gather16(v, jnp.zeros((L,), jnp.int32) + i)


def _eq16(a, b):
    d = a ^ b
    return 1 - (((d | (0 - d)) >> 31) & 1)


def _ge16(a, b):
    return 1 - (((a - b) >> 31) & 1)


def _cumsum16(x):
    lanes = jax.lax.iota(jnp.int32, L)
    y = x
    for s in (1, 2, 4, 8):
        g = _gather16(y, jnp.maximum(lanes - s, 0))
        m = ((s - 1 - lanes) >> 31) & 1
        y = y + g * m
    return y


def _dispatch_sc(x, e0, e1, counts):
    mesh = plsc.VectorSubcoreMesh(core_axis_name="c", subcore_axis_name="s")

    @functools.partial(
        pl.kernel,
        out_type=[
            jax.ShapeDtypeStruct((NPAD, H), jnp.float32),
            jax.ShapeDtypeStruct((T,), jnp.int32),
            jax.ShapeDtypeStruct((T,), jnp.int32),
            jax.ShapeDtypeStruct((MLEN,), jnp.int32),
        ],
        mesh=mesh,
        scratch_types=[
            pltpu.VMEM((TW,), jnp.int32),
            pltpu.VMEM((TW,), jnp.int32),
            pltpu.VMEM((TW,), jnp.int32),
            pltpu.VMEM((TW,), jnp.int32),
            pltpu.VMEM((TW, H), jnp.float32),
            pltpu.VMEM((NW, E), jnp.int32),
            pltpu.VMEM((MLEN,), jnp.int32),
            pltpu.SemaphoreType.DMA,
            pltpu.SemaphoreType.DMA,
        ],
    )
    def k(x_hbm, e0_hbm, e1_hbm, c_hbm, xs_hbm, p0_hbm, p1_hbm, meta_hbm,
          e0v, e1v, p0v, p1v, xrows, cnts, metav, semx, sems):
        wid = lax.axis_index("s") * NC + lax.axis_index("c")
        tb = wid * TW
        cpx = pltpu.async_copy(x_hbm.at[pl.ds(tb, TW)], xrows, semx)
        pltpu.sync_copy(e0_hbm.at[pl.ds(tb, TW)], e0v)
        pltpu.sync_copy(e1_hbm.at[pl.ds(tb, TW)], e1v)
        pltpu.sync_copy(c_hbm, cnts)

        lanes = jax.lax.iota(jnp.int32, L)
        totals = jnp.zeros((L,), jnp.int32)
        baserel = jnp.zeros((L,), jnp.int32)
        for w2 in range(NW):
            row = cnts[w2]
            totals = totals + row
            baserel = baserel + row * (((w2 - wid) >> 31) & 1)

        nblk = (totals + (B - 1)) >> 6
        cblk = _cumsum16(nblk)
        offb = cblk - nblk
        base = (offb << 6) + baserel

        for c in range(TW // L):
            ev0 = e0v[pl.ds(c * L, L)]
            ev1 = e1v[pl.ds(c * L, L)]
            p0a = jnp.zeros((L,), jnp.int32)
            p1a = jnp.zeros((L,), jnp.int32)
            for e in range(E):
                b_e = _splat16(base, e)
                m0i = _eq16(ev0, jnp.full((L,), e, jnp.int32))
                cs0 = _cumsum16(m0i)
                p0a = p0a + (b_e + cs0 - 1 - p0a) * m0i
                n0 = _splat16(cs0, L - 1)
                m1i = _eq16(ev1, jnp.full((L,), e, jnp.int32))
                cs1 = _cumsum16(m1i)
                p1a = p1a + (b_e + n0 + cs1 - 1 - p1a) * m1i
                n1 = _splat16(cs1, L - 1)
                base = base + _eq16(lanes, jnp.full((L,), e, jnp.int32)) * (n0 + n1)
            p0v[pl.ds(c * L, L)] = p0a
            p1v[pl.ds(c * L, L)] = p1a

        pltpu.sync_copy(p0v, p0_hbm.at[pl.ds(tb, TW)])
        pltpu.sync_copy(p1v, p1_hbm.at[pl.ds(tb, TW)])
        cpx.wait()
        pltpu.async_copy(xrows, xs_hbm.at[p0v], sems).wait()
        pltpu.async_copy(xrows, xs_hbm.at[p1v], sems).wait()

        @pl.when(wid == 0)
        def _():
            used = _splat16(cblk, L - 1)
            for k6 in range(MLEN // L):
                biota = lanes + L * k6
                acc = jnp.full((L,), -1, jnp.int32)
                for e in range(E):
                    ob = _splat16(offb, e)
                    acc = acc + _ge16(biota, ob)
                acc = jnp.minimum(acc, E - 1)
                if k6 == NBLK // L:
                    acc = acc + (used - acc) * _eq16(
                        lanes, jnp.full((L,), NBLK % L, jnp.int32))
                metav[pl.ds(k6 * L, L)] = acc
            pltpu.sync_copy(metav, meta_hbm)

    return k(x, e0, e1, counts)


def _gmm_body(meta_ref, x_ref, wgu_ref, wd_ref, y_ref, wgu_b, wd_b):
    i = pl.program_id(0)
    live = i < meta_ref[NBLK]
    changed = jnp.logical_or(i == 0, meta_ref[i] != meta_ref[jnp.maximum(i - 1, 0)])

    @pl.when(jnp.logical_and(live, changed))
    def _():
        wgu_b[...] = wgu_ref[0].astype(jnp.bfloat16)
        wd_b[...] = wd_ref[0].astype(jnp.bfloat16)

    @pl.when(live)
    def _():
        h = jnp.dot(x_ref[...].astype(jnp.bfloat16), wgu_b[...],
                    preferred_element_type=jnp.float32)
        a = h[:, :I]
        b = h[:, I:]
        g = a * jax.nn.sigmoid(a) * b
        y_ref[...] = jnp.dot(g.astype(jnp.bfloat16), wd_b[...],
                             preferred_element_type=jnp.float32)


def _grouped_matmul(x_sorted, wgu, wd, meta):
    grid_spec = pltpu.PrefetchScalarGridSpec(
        num_scalar_prefetch=1,
        grid=(NBLK,),
        in_specs=[
            pl.BlockSpec((B, H), lambda i, m: (i, 0)),
            pl.BlockSpec((1, H, 2 * I), lambda i, m: (m[i], 0, 0)),
            pl.BlockSpec((1, I, H), lambda i, m: (m[i], 0, 0)),
        ],
        out_specs=pl.BlockSpec((B, H), lambda i, m: (i, 0)),
        scratch_shapes=[
            pltpu.VMEM((H, 2 * I), jnp.bfloat16),
            pltpu.VMEM((I, H), jnp.bfloat16),
        ],
    )
    return pl.pallas_call(
        _gmm_body,
        grid_spec=grid_spec,
        out_shape=jax.ShapeDtypeStruct((NPAD, H), jnp.float32),
    )(meta, x_sorted, wgu, wd)


def _combine_sc(y, pos0, pos1, w0, w1, sh):
    mesh = plsc.VectorSubcoreMesh(core_axis_name="c", subcore_axis_name="s")

    @functools.partial(
        pl.kernel,
        out_type=jax.ShapeDtypeStruct((T, H), jnp.float32),
        mesh=mesh,
        scratch_types=[
            pltpu.VMEM((L,), jnp.int32),
            pltpu.VMEM((L,), jnp.int32),
            pltpu.VMEM((L, H), jnp.float32),
            pltpu.VMEM((L, H), jnp.float32),
            pltpu.VMEM((L, H), jnp.float32),
            pltpu.VMEM((L,), jnp.float32),
            pltpu.VMEM((L,), jnp.float32),
            pltpu.SemaphoreType.DMA,
            pltpu.SemaphoreType.DMA,
        ],
    )
    def k(y_hbm, p0_hbm, p1_hbm, w0_hbm, w1_hbm, sh_hbm, o_hbm,
          idx0, idx1, y0, y1, shv, w0v, w1v, s0, s1):
        wid = lax.axis_index("s") * NC + lax.axis_index("c")
        for c in range(TW // L):
            tb = wid * TW + c * L
            pltpu.sync_copy(p0_hbm.at[pl.ds(tb, L)], idx0)
            pltpu.sync_copy(p1_hbm.at[pl.ds(tb, L)], idx1)
            cp0 = pltpu.async_copy(y_hbm.at[idx0], y0, s0)
            cp1 = pltpu.async_copy(y_hbm.at[idx1], y1, s1)
            pltpu.sync_copy(sh_hbm.at[pl.ds(tb, L)], shv)
            pltpu.sync_copy(w0_hbm.at[pl.ds(tb, L)], w0v)
            pltpu.sync_copy(w1_hbm.at[pl.ds(tb, L)], w1v)
            cp0.wait()
            cp1.wait()
            w0all = w0v[...] * SCALE
            w1all = w1v[...] * SCALE

            def body_l(l, carry):
                a0 = _splat16(w0all, l)
                a1 = _splat16(w1all, l)
                for kk in range(H // L):
                    sl = pl.ds(kk * L, L)
                    shv[l, sl] = shv[l, sl] + a0 * y0[l, sl] + a1 * y1[l, sl]
                return carry

            lax.fori_loop(0, L, body_l, 0)
            pltpu.sync_copy(shv, o_hbm.at[pl.ds(tb, L)])

    return k(y, pos0, pos1, w0, w1, sh)


def kernel(hidden_states, gate_w, w_gate_up, w_down, shared_gate_up, shared_down):
    x = hidden_states.reshape(T, H)
    shared_out, e0, e1, w0, w1, counts = _shared_and_router(
        x, shared_gate_up, shared_down, gate_w)
    e0 = e0.reshape(T)
    e1 = e1.reshape(T)
    counts = counts.reshape(NW, E)

    xs, pos0, pos1, meta = _dispatch_sc(x, e0, e1, counts)
    y = _grouped_matmul(xs, w_gate_up, w_down, meta)
    out = _combine_sc(y, pos0, pos1, w0.reshape(T), w1.reshape(T), shared_out)
    return out.reshape(T, H)


# speedup vs baseline: 1.0829x; 1.0829x over previous
import functools

import jax
import jax.numpy as jnp
from jax import lax
from jax.experimental import pallas as pl
from jax.experimental.pallas import tpu as pltpu
from jax.experimental.pallas import tpu_sc as plsc

T = 2048
H = 1024
E = 16
I = 512
TOPK = 2
NG = 4
TG = 2
ISH = 1024
SCALE = 1.0

TBLK = 256
B = 64
NPAD = T * TOPK + E * B
NBLK = NPAD // B
MLEN = 96

NC = 2
NS = 16
NW = NC * NS
TW = T // NW
L = 16


def _shared_router_body(x_ref, sgu_ref, sd_ref, gw_ref,
                        so_ref, e0_ref, e1_ref, w0_ref, w1_ref, cnt_ref):
    x = x_ref[...]
    h = jnp.dot(x, sgu_ref[...], preferred_element_type=jnp.float32)
    a = h[:, :ISH]
    b = h[:, ISH:]
    g = a * jax.nn.sigmoid(a) * b
    so_ref[...] = jnp.dot(g, sd_ref[...], preferred_element_type=jnp.float32)

    logits = jax.lax.dot_general(
        x, gw_ref[...], (((1,), (1,)), ((), ())),
        preferred_element_type=jnp.float32)
    m = jnp.max(logits, axis=1, keepdims=True)
    ex = jnp.exp(logits - m)
    sc = ex / jnp.sum(ex, axis=1, keepdims=True)

    gs = [jnp.max(sc[:, 4 * k:4 * k + 4], axis=1, keepdims=True)
          for k in range(NG)]
    col = jax.lax.broadcasted_iota(jnp.int32, (TBLK, E), 1)
    colg = col // (E // NG)
    masked = sc
    for j in range(NG):
        r = jnp.zeros((TBLK, 1), jnp.int32)
        for k in range(NG):
            gt = gs[k] > gs[j]
            tie = (gs[k] == gs[j]) & (k < j)
            r = r + jnp.where(gt | tie, 1, 0)
        keep = r < TG
        masked = jnp.where((colg == j) & jnp.logical_not(keep), 0.0, masked)

    m0 = jnp.max(masked, axis=1, keepdims=True)
    i0 = jnp.min(jnp.where(masked == m0, col, E), axis=1, keepdims=True)
    masked2 = jnp.where(col == i0, -1.0, masked)
    m1 = jnp.max(masked2, axis=1, keepdims=True)
    i1 = jnp.min(jnp.where(masked2 == m1, col, E), axis=1, keepdims=True)
    s = m0 + m1 + 1e-20
    e0_ref[...] = i0
    e1_ref[...] = i1
    w0_ref[...] = m0 / s
    w1_ref[...] = m1 / s

    for sub in range(TBLK // TW):
        s0 = i0[sub * TW:(sub + 1) * TW]
        s1 = i1[sub * TW:(sub + 1) * TW]
        cole = jax.lax.broadcasted_iota(jnp.int32, (TW, E), 1)
        cnt = (jnp.sum(jnp.where(s0 == cole, 1, 0), axis=0, keepdims=True)
               + jnp.sum(jnp.where(s1 == cole, 1, 0), axis=0, keepdims=True))
        cnt_ref[sub] = cnt


def _shared_and_router(x, sgu, sd, gw):
    grid = (T // TBLK,)
    return pl.pallas_call(
        _shared_router_body,
        grid=grid,
        in_specs=[
            pl.BlockSpec((TBLK, H), lambda i: (i, 0)),
            pl.BlockSpec((H, 2 * ISH), lambda i: (0, 0)),
            pl.BlockSpec((ISH, H), lambda i: (0, 0)),
            pl.BlockSpec((E, H), lambda i: (0, 0)),
        ],
        out_specs=[
            pl.BlockSpec((TBLK, H), lambda i: (i, 0)),
            pl.BlockSpec((TBLK, 1), lambda i: (i, 0)),
            pl.BlockSpec((TBLK, 1), lambda i: (i, 0)),
            pl.BlockSpec((TBLK, 1), lambda i: (i, 0)),
            pl.BlockSpec((TBLK, 1), lambda i: (i, 0)),
            pl.BlockSpec((TBLK // TW, 1, E), lambda i: (i, 0, 0)),
        ],
        out_shape=[
            jax.ShapeDtypeStruct((T, H), jnp.float32),
            jax.ShapeDtypeStruct((T, 1), jnp.int32),
            jax.ShapeDtypeStruct((T, 1), jnp.int32),
            jax.ShapeDtypeStruct((T, 1), jnp.float32),
            jax.ShapeDtypeStruct((T, 1), jnp.float32),
            jax.ShapeDtypeStruct((NW, 1, E), jnp.int32),
        ],
    )(x, sgu, sd, gw)


_GDN = jax.lax.GatherDimensionNumbers(
    offset_dims=(), collapsed_slice_dims=(0,), start_index_map=(0,))


def _gather16(v, idx):
    return jax.lax.gather(
        v, idx.reshape(L, 1), _GDN, (1,),
        mode=jax.lax.GatherScatterMode.PROMISE_IN_BOUNDS)


def _splat16(v, i):
    return _gather16(v, jnp.zeros((L,), jnp.int32) + i)


def _eq16(a, b):
    d = a ^ b
    return 1 - (((d | (0 - d)) >> 31) & 1)


def _ge16(a, b):
    return 1 - (((a - b) >> 31) & 1)


def _cumsum16(x):
    lanes = jax.lax.iota(jnp.int32, L)
    y = x
    for s in (1, 2, 4, 8):
        g = _gather16(y, jnp.maximum(lanes - s, 0))
        m = ((s - 1 - lanes) >> 31) & 1
        y = y + g * m
    return y


def _dispatch_sc(x, e0, e1, counts):
    mesh = plsc.VectorSubcoreMesh(core_axis_name="c", subcore_axis_name="s")

    @functools.partial(
        pl.kernel,
        out_type=[
            jax.ShapeDtypeStruct((NPAD, H), jnp.float32),
            jax.ShapeDtypeStruct((T,), jnp.int32),
            jax.ShapeDtypeStruct((T,), jnp.int32),
            jax.ShapeDtypeStruct((MLEN,), jnp.int32),
        ],
        mesh=mesh,
        scratch_types=[
            pltpu.VMEM((TW,), jnp.int32),
            pltpu.VMEM((TW,), jnp.int32),
            pltpu.VMEM((TW,), jnp.int32),
            pltpu.VMEM((TW,), jnp.int32),
            pltpu.VMEM((TW, H), jnp.float32),
            pltpu.VMEM((NW, E), jnp.int32),
            pltpu.VMEM((MLEN,), jnp.int32),
            pltpu.SemaphoreType.DMA,
            pltpu.SemaphoreType.DMA,
        ],
    )
    def k(x_hbm, e0_hbm, e1_hbm, c_hbm, xs_hbm, p0_hbm, p1_hbm, meta_hbm,
          e0v, e1v, p0v, p1v, xrows, cnts, metav, semx, sems):
        wid = lax.axis_index("s") * NC + lax.axis_index("c")
        tb = wid * TW
        cpx = pltpu.async_copy(x_hbm.at[pl.ds(tb, TW)], xrows, semx)
        pltpu.sync_copy(e0_hbm.at[pl.ds(tb, TW)], e0v)
        pltpu.sync_copy(e1_hbm.at[pl.ds(tb, TW)], e1v)
        pltpu.sync_copy(c_hbm, cnts)

        lanes = jax.lax.iota(jnp.int32, L)
        totals = jnp.zeros((L,), jnp.int32)
        baserel = jnp.zeros((L,), jnp.int32)
        for w2 in range(NW):
            row = cnts[w2]
            totals = totals + row
            baserel = baserel + row * (((w2 - wid) >> 31) & 1)

        nblk = (totals + (B - 1)) >> 6
        cblk = _cumsum16(nblk)
        offb = cblk - nblk
        base = (offb << 6) + baserel

        for c in range(TW // L):
            ev0 = e0v[pl.ds(c * L, L)]
            ev1 = e1v[pl.ds(c * L, L)]
            p0a = jnp.zeros((L,), jnp.int32)
            p1a = jnp.zeros((L,), jnp.int32)
            for e in range(E):
                b_e = _splat16(base, e)
                m0i = _eq16(ev0, jnp.full((L,), e, jnp.int32))
                cs0 = _cumsum16(m0i)
                p0a = p0a + (b_e + cs0 - 1 - p0a) * m0i
                n0 = _splat16(cs0, L - 1)
                m1i = _eq16(ev1, jnp.full((L,), e, jnp.int32))
                cs1 = _cumsum16(m1i)
                p1a = p1a + (b_e + n0 + cs1 - 1 - p1a) * m1i
                n1 = _splat16(cs1, L - 1)
                base = base + _eq16(lanes, jnp.full((L,), e, jnp.int32)) * (n0 + n1)
            p0v[pl.ds(c * L, L)] = p0a
            p1v[pl.ds(c * L, L)] = p1a

        pltpu.sync_copy(p0v, p0_hbm.at[pl.ds(tb, TW)])
        pltpu.sync_copy(p1v, p1_hbm.at[pl.ds(tb, TW)])
        cpx.wait()
        cpa = pltpu.async_copy(xrows, xs_hbm.at[p0v], sems)
        cpb = pltpu.async_copy(xrows, xs_hbm.at[p1v], semx)
        cpa.wait()
        cpb.wait()

        @pl.when(wid == 0)
        def _():
            used = _splat16(cblk, L - 1)
            for k6 in range(MLEN // L):
                biota = lanes + L * k6
                acc = jnp.full((L,), -1, jnp.int32)
                for e in range(E):
                    ob = _splat16(offb, e)
                    acc = acc + _ge16(biota, ob)
                acc = jnp.minimum(acc, E - 1)
                if k6 == NBLK // L:
                    acc = acc + (used - acc) * _eq16(
                        lanes, jnp.full((L,), NBLK % L, jnp.int32))
                metav[pl.ds(k6 * L, L)] = acc
            pltpu.sync_copy(metav, meta_hbm)

    return k(x, e0, e1, counts)


def _gmm_body(meta_ref, x_ref, wgu_ref, wd_ref, y_ref):
    i = pl.program_id(0)

    @pl.when(i < meta_ref[NBLK])
    def _():
        h = jnp.dot(x_ref[...], wgu_ref[0], preferred_element_type=jnp.float32)
        a = h[:, :I]
        b = h[:, I:]
        g = a * jax.nn.sigmoid(a) * b
        y_ref[...] = jnp.dot(g, wd_ref[0], preferred_element_type=jnp.float32)


def _grouped_matmul(x_sorted, wgu, wd, meta):
    grid_spec = pltpu.PrefetchScalarGridSpec(
        num_scalar_prefetch=1,
        grid=(NBLK,),
        in_specs=[
            pl.BlockSpec((B, H), lambda i, m: (i, 0)),
            pl.BlockSpec((1, H, 2 * I), lambda i, m: (m[i], 0, 0)),
            pl.BlockSpec((1, I, H), lambda i, m: (m[i], 0, 0)),
        ],
        out_specs=pl.BlockSpec((B, H), lambda i, m: (i, 0)),
    )
    return pl.pallas_call(
        _gmm_body,
        grid_spec=grid_spec,
        out_shape=jax.ShapeDtypeStruct((NPAD, H), jnp.float32),
    )(meta, x_sorted, wgu, wd)


def _combine_sc(y, pos0, pos1, w0, w1, sh):
    mesh = plsc.VectorSubcoreMesh(core_axis_name="c", subcore_axis_name="s")

    @functools.partial(
        pl.kernel,
        out_type=jax.ShapeDtypeStruct((T, H), jnp.float32),
        mesh=mesh,
        scratch_types=[
            pltpu.VMEM((2, L), jnp.int32),
            pltpu.VMEM((2, L), jnp.int32),
            pltpu.VMEM((2, L, H), jnp.float32),
            pltpu.VMEM((2, L, H), jnp.float32),
            pltpu.VMEM((2, L, H), jnp.float32),
            pltpu.VMEM((2, L), jnp.float32),
            pltpu.VMEM((2, L), jnp.float32),
            pltpu.SemaphoreType.DMA,
            pltpu.SemaphoreType.DMA,
            pltpu.SemaphoreType.DMA,
            pltpu.SemaphoreType.DMA,
            pltpu.SemaphoreType.DMA,
            pltpu.SemaphoreType.DMA,
            pltpu.SemaphoreType.DMA,
            pltpu.SemaphoreType.DMA,
            pltpu.SemaphoreType.DMA,
            pltpu.SemaphoreType.DMA,
        ],
    )
    def k(y_hbm, p0_hbm, p1_hbm, w0_hbm, w1_hbm, sh_hbm, o_hbm,
          idx0, idx1, y0, y1, shv, w0v, w1v,
          sy0a, sy1a, ssha, swa, soa, sy0b, sy1b, sshb, swb, sob):
        wid = lax.axis_index("s") * NC + lax.axis_index("c")
        nch = TW // L
        sems = [(sy0a, sy1a, ssha, swa, soa), (sy0b, sy1b, sshb, swb, sob)]

        def issue(c, bi):
            tb = wid * TW + c * L
            sy0, sy1, ssh, sw, _ = sems[bi]
            pltpu.sync_copy(p0_hbm.at[pl.ds(tb, L)], idx0.at[bi])
            pltpu.sync_copy(p1_hbm.at[pl.ds(tb, L)], idx1.at[bi])
            cps = (pltpu.async_copy(y_hbm.at[idx0.at[bi]], y0.at[bi], sy0),
                   pltpu.async_copy(y_hbm.at[idx1.at[bi]], y1.at[bi], sy1),
                   pltpu.async_copy(sh_hbm.at[pl.ds(tb, L)], shv.at[bi], ssh),
                   pltpu.async_copy(w0_hbm.at[pl.ds(tb, L)], w0v.at[bi], sw),
                   pltpu.async_copy(w1_hbm.at[pl.ds(tb, L)], w1v.at[bi], sw))
            return cps

        pend = issue(0, 0)
        owrite = None
        for c in range(nch):
            bi = c % 2
            if c + 1 < nch:
                if owrite is not None:
                    owrite.wait()
                    owrite = None
                nxt = issue(c + 1, 1 - bi)
            for cp in pend:
                cp.wait()
            if owrite is not None:
                owrite.wait()
                owrite = None
            w0all = w0v[bi] * SCALE
            w1all = w1v[bi] * SCALE

            def body_l(l, carry):
                a0 = _splat16(w0all, l)
                a1 = _splat16(w1all, l)
                for kk in range(H // L):
                    sl = pl.ds(kk * L, L)
                    shv[bi, l, sl] = (shv[bi, l, sl]
                                      + a0 * y0[bi, l, sl] + a1 * y1[bi, l, sl])
                return carry

            lax.fori_loop(0, L, body_l, 0)
            tb = wid * TW + c * L
            owrite = pltpu.async_copy(shv.at[bi], o_hbm.at[pl.ds(tb, L)],
                                      sems[bi][4])
            if c + 1 < nch:
                pend = nxt
        owrite.wait()

    return k(y, pos0, pos1, w0, w1, sh)


def kernel(hidden_states, gate_w, w_gate_up, w_down, shared_gate_up, shared_down):
    x = hidden_states.reshape(T, H)
    shared_out, e0, e1, w0, w1, counts = _shared_and_router(
        x, shared_gate_up, shared_down, gate_w)
    e0 = e0.reshape(T)
    e1 = e1.reshape(T)
    counts = counts.reshape(NW, E)

    xs, pos0, pos1, meta = _dispatch_sc(x, e0, e1, counts)
    y = _grouped_matmul(xs, w_gate_up, w_down, meta)
    out = _combine_sc(y, pos0, pos1, w0.reshape(T), w1.reshape(T), shared_out)
    return out.reshape(T, H)


# speedup vs baseline: 1.1051x; 1.0205x over previous
import functools

import jax
import jax.numpy as jnp
from jax import lax
from jax.experimental import pallas as pl
from jax.experimental.pallas import tpu as pltpu
from jax.experimental.pallas import tpu_sc as plsc

T = 2048
H = 1024
E = 16
I = 512
TOPK = 2
NG = 4
TG = 2
ISH = 1024
SCALE = 1.0

TBLK = 256
B = 64
NPAD = T * TOPK + E * B
NBLK = NPAD // B
MLEN = 96

NC = 2
NS = 16
NW = NC * NS
TW = T // NW
L = 16


def _shared_router_body(x_ref, sgu_ref, sd_ref, gw_ref,
                        so_ref, e0_ref, e1_ref, w0_ref, w1_ref, cnt_ref):
    x = x_ref[...]
    h = jnp.dot(x, sgu_ref[...], preferred_element_type=jnp.float32)
    a = h[:, :ISH]
    b = h[:, ISH:]
    g = a * jax.nn.sigmoid(a) * b
    so_ref[...] = jnp.dot(g, sd_ref[...], preferred_element_type=jnp.float32)

    logits = jax.lax.dot_general(
        x, gw_ref[...], (((1,), (1,)), ((), ())),
        preferred_element_type=jnp.float32)
    m = jnp.max(logits, axis=1, keepdims=True)
    ex = jnp.exp(logits - m)
    sc = ex / jnp.sum(ex, axis=1, keepdims=True)

    gs = [jnp.max(sc[:, 4 * k:4 * k + 4], axis=1, keepdims=True)
          for k in range(NG)]
    col = jax.lax.broadcasted_iota(jnp.int32, (TBLK, E), 1)
    colg = col // (E // NG)
    masked = sc
    for j in range(NG):
        r = jnp.zeros((TBLK, 1), jnp.int32)
        for k in range(NG):
            gt = gs[k] > gs[j]
            tie = (gs[k] == gs[j]) & (k < j)
            r = r + jnp.where(gt | tie, 1, 0)
        keep = r < TG
        masked = jnp.where((colg == j) & jnp.logical_not(keep), 0.0, masked)

    m0 = jnp.max(masked, axis=1, keepdims=True)
    i0 = jnp.min(jnp.where(masked == m0, col, E), axis=1, keepdims=True)
    masked2 = jnp.where(col == i0, -1.0, masked)
    m1 = jnp.max(masked2, axis=1, keepdims=True)
    i1 = jnp.min(jnp.where(masked2 == m1, col, E), axis=1, keepdims=True)
    s = m0 + m1 + 1e-20
    e0_ref[...] = i0
    e1_ref[...] = i1
    w0_ref[...] = m0 / s
    w1_ref[...] = m1 / s

    for sub in range(TBLK // TW):
        s0 = i0[sub * TW:(sub + 1) * TW]
        s1 = i1[sub * TW:(sub + 1) * TW]
        cole = jax.lax.broadcasted_iota(jnp.int32, (TW, E), 1)
        cnt = (jnp.sum(jnp.where(s0 == cole, 1, 0), axis=0, keepdims=True)
               + jnp.sum(jnp.where(s1 == cole, 1, 0), axis=0, keepdims=True))
        cnt_ref[sub] = cnt


def _shared_and_router(x, sgu, sd, gw):
    grid = (T // TBLK,)
    return pl.pallas_call(
        _shared_router_body,
        grid=grid,
        in_specs=[
            pl.BlockSpec((TBLK, H), lambda i: (i, 0)),
            pl.BlockSpec((H, 2 * ISH), lambda i: (0, 0)),
            pl.BlockSpec((ISH, H), lambda i: (0, 0)),
            pl.BlockSpec((E, H), lambda i: (0, 0)),
        ],
        out_specs=[
            pl.BlockSpec((TBLK, H), lambda i: (i, 0)),
            pl.BlockSpec((TBLK, 1), lambda i: (i, 0)),
            pl.BlockSpec((TBLK, 1), lambda i: (i, 0)),
            pl.BlockSpec((TBLK, 1), lambda i: (i, 0)),
            pl.BlockSpec((TBLK, 1), lambda i: (i, 0)),
            pl.BlockSpec((TBLK // TW, 1, E), lambda i: (i, 0, 0)),
        ],
        out_shape=[
            jax.ShapeDtypeStruct((T, H), jnp.float32),
            jax.ShapeDtypeStruct((T, 1), jnp.int32),
            jax.ShapeDtypeStruct((T, 1), jnp.int32),
            jax.ShapeDtypeStruct((T, 1), jnp.float32),
            jax.ShapeDtypeStruct((T, 1), jnp.float32),
            jax.ShapeDtypeStruct((NW, 1, E), jnp.int32),
        ],
    )(x, sgu, sd, gw)


_GDN = jax.lax.GatherDimensionNumbers(
    offset_dims=(), collapsed_slice_dims=(0,), start_index_map=(0,))


def _gather16(v, idx):
    return jax.lax.gather(
        v, idx.reshape(L, 1), _GDN, (1,),
        mode=jax.lax.GatherScatterMode.PROMISE_IN_BOUNDS)


def _splat16(v, i):
    return _gather16(v, jnp.zeros((L,), jnp.int32) + i)


def _eq16(a, b):
    d = a ^ b
    return 1 - (((d | (0 - d)) >> 31) & 1)


def _ge16(a, b):
    return 1 - (((a - b) >> 31) & 1)


def _cumsum16(x):
    lanes = jax.lax.iota(jnp.int32, L)
    y = x
    for s in (1, 2, 4, 8):
        g = _gather16(y, jnp.maximum(lanes - s, 0))
        m = ((s - 1 - lanes) >> 31) & 1
        y = y + g * m
    return y


def _dispatch_sc(x, e0, e1, counts):
    mesh = plsc.VectorSubcoreMesh(core_axis_name="c", subcore_axis_name="s")

    @functools.partial(
        pl.kernel,
        out_type=[
            jax.ShapeDtypeStruct((NPAD, H), jnp.float32),
            jax.ShapeDtypeStruct((T,), jnp.int32),
            jax.ShapeDtypeStruct((T,), jnp.int32),
            jax.ShapeDtypeStruct((MLEN,), jnp.int32),
        ],
        mesh=mesh,
        scratch_types=[
            pltpu.VMEM((TW,), jnp.int32),
            pltpu.VMEM((TW,), jnp.int32),
            pltpu.VMEM((TW,), jnp.int32),
            pltpu.VMEM((TW,), jnp.int32),
            pltpu.VMEM((TW, H), jnp.float32),
            pltpu.VMEM((NW, E), jnp.int32),
            pltpu.VMEM((MLEN,), jnp.int32),
            pltpu.SemaphoreType.DMA,
            pltpu.SemaphoreType.DMA,
        ],
    )
    def k(x_hbm, e0_hbm, e1_hbm, c_hbm, xs_hbm, p0_hbm, p1_hbm, meta_hbm,
          e0v, e1v, p0v, p1v, xrows, cnts, metav, semx, sems):
        wid = lax.axis_index("s") * NC + lax.axis_index("c")
        tb = wid * TW
        cpx = pltpu.async_copy(x_hbm.at[pl.ds(tb, TW)], xrows, semx)
        pltpu.sync_copy(e0_hbm.at[pl.ds(tb, TW)], e0v)
        pltpu.sync_copy(e1_hbm.at[pl.ds(tb, TW)], e1v)
        pltpu.sync_copy(c_hbm, cnts)

        lanes = jax.lax.iota(jnp.int32, L)
        totals = jnp.zeros((L,), jnp.int32)
        baserel = jnp.zeros((L,), jnp.int32)
        for w2 in range(NW):
            row = cnts[w2]
            totals = totals + row
            baserel = baserel + row * (((w2 - wid) >> 31) & 1)

        nblk = (totals + (B - 1)) >> 6
        cblk = _cumsum16(nblk)
        offb = cblk - nblk
        base = (offb << 6) + baserel

        for c in range(TW // L):
            ev0 = e0v[pl.ds(c * L, L)]
            ev1 = e1v[pl.ds(c * L, L)]
            p0a = jnp.zeros((L,), jnp.int32)
            p1a = jnp.zeros((L,), jnp.int32)
            for e in range(E):
                b_e = _splat16(base, e)
                m0i = _eq16(ev0, jnp.full((L,), e, jnp.int32))
                cs0 = _cumsum16(m0i)
                p0a = p0a + (b_e + cs0 - 1 - p0a) * m0i
                n0 = _splat16(cs0, L - 1)
                m1i = _eq16(ev1, jnp.full((L,), e, jnp.int32))
                cs1 = _cumsum16(m1i)
                p1a = p1a + (b_e + n0 + cs1 - 1 - p1a) * m1i
                n1 = _splat16(cs1, L - 1)
                base = base + _eq16(lanes, jnp.full((L,), e, jnp.int32)) * (n0 + n1)
            p0v[pl.ds(c * L, L)] = p0a
            p1v[pl.ds(c * L, L)] = p1a

        pltpu.sync_copy(p0v, p0_hbm.at[pl.ds(tb, TW)])
        pltpu.sync_copy(p1v, p1_hbm.at[pl.ds(tb, TW)])
        cpx.wait()
        cpa = pltpu.async_copy(xrows, xs_hbm.at[p0v], sems)
        cpb = pltpu.async_copy(xrows, xs_hbm.at[p1v], semx)
        cpa.wait()
        cpb.wait()

        @pl.when(wid == 0)
        def _():
            used = _splat16(cblk, L - 1)
            for k6 in range(MLEN // L):
                biota = lanes + L * k6
                acc = jnp.full((L,), -1, jnp.int32)
                for e in range(E):
                    ob = _splat16(offb, e)
                    acc = acc + _ge16(biota, ob)
                acc = jnp.minimum(acc, E - 1)
                if k6 == NBLK // L:
                    acc = acc + (used - acc) * _eq16(
                        lanes, jnp.full((L,), NBLK % L, jnp.int32))
                metav[pl.ds(k6 * L, L)] = acc
            pltpu.sync_copy(metav, meta_hbm)

    return k(x, e0, e1, counts)


def _gmm_body(meta_ref, x_ref, wgu_ref, wd_ref, y_ref):
    i = pl.program_id(0)

    @pl.when(i < meta_ref[NBLK])
    def _():
        h = jnp.dot(x_ref[...], wgu_ref[0], preferred_element_type=jnp.float32)
        a = h[:, :I]
        b = h[:, I:]
        g = a * jax.nn.sigmoid(a) * b
        y_ref[...] = jnp.dot(g, wd_ref[0], preferred_element_type=jnp.float32)


def _grouped_matmul(x_sorted, wgu, wd, meta):
    grid_spec = pltpu.PrefetchScalarGridSpec(
        num_scalar_prefetch=1,
        grid=(NBLK,),
        in_specs=[
            pl.BlockSpec((B, H),
                         lambda i, m: (jnp.where(i < m[NBLK], i, NBLK - 1), 0)),
            pl.BlockSpec((1, H, 2 * I), lambda i, m: (m[i], 0, 0)),
            pl.BlockSpec((1, I, H), lambda i, m: (m[i], 0, 0)),
        ],
        out_specs=pl.BlockSpec(
            (B, H), lambda i, m: (jnp.where(i < m[NBLK], i, NBLK - 1), 0)),
    )
    return pl.pallas_call(
        _gmm_body,
        grid_spec=grid_spec,
        out_shape=jax.ShapeDtypeStruct((NPAD, H), jnp.float32),
    )(meta, x_sorted, wgu, wd)


def _combine_sc(y, pos0, pos1, w0, w1, sh):
    mesh = plsc.VectorSubcoreMesh(core_axis_name="c", subcore_axis_name="s")

    @functools.partial(
        pl.kernel,
        out_type=jax.ShapeDtypeStruct((T, H), jnp.float32),
        mesh=mesh,
        scratch_types=[
            pltpu.VMEM((2, L), jnp.int32),
            pltpu.VMEM((2, L), jnp.int32),
            pltpu.VMEM((2, L, H), jnp.float32),
            pltpu.VMEM((2, L, H), jnp.float32),
            pltpu.VMEM((2, L, H), jnp.float32),
            pltpu.VMEM((2, L), jnp.float32),
            pltpu.VMEM((2, L), jnp.float32),
            pltpu.SemaphoreType.DMA,
            pltpu.SemaphoreType.DMA,
            pltpu.SemaphoreType.DMA,
            pltpu.SemaphoreType.DMA,
            pltpu.SemaphoreType.DMA,
            pltpu.SemaphoreType.DMA,
            pltpu.SemaphoreType.DMA,
            pltpu.SemaphoreType.DMA,
            pltpu.SemaphoreType.DMA,
            pltpu.SemaphoreType.DMA,
        ],
    )
    def k(y_hbm, p0_hbm, p1_hbm, w0_hbm, w1_hbm, sh_hbm, o_hbm,
          idx0, idx1, y0, y1, shv, w0v, w1v,
          sy0a, sy1a, ssha, swa, soa, sy0b, sy1b, sshb, swb, sob):
        wid = lax.axis_index("s") * NC + lax.axis_index("c")
        nch = TW // L
        sems = [(sy0a, sy1a, ssha, swa, soa), (sy0b, sy1b, sshb, swb, sob)]

        def issue(c, bi):
            tb = wid * TW + c * L
            sy0, sy1, ssh, sw, _ = sems[bi]
            pltpu.sync_copy(p0_hbm.at[pl.ds(tb, L)], idx0.at[bi])
            pltpu.sync_copy(p1_hbm.at[pl.ds(tb, L)], idx1.at[bi])
            cps = (pltpu.async_copy(y_hbm.at[idx0.at[bi]], y0.at[bi], sy0),
                   pltpu.async_copy(y_hbm.at[idx1.at[bi]], y1.at[bi], sy1),
                   pltpu.async_copy(sh_hbm.at[pl.ds(tb, L)], shv.at[bi], ssh),
                   pltpu.async_copy(w0_hbm.at[pl.ds(tb, L)], w0v.at[bi], sw),
                   pltpu.async_copy(w1_hbm.at[pl.ds(tb, L)], w1v.at[bi], sw))
            return cps

        pend = issue(0, 0)
        owrite = None
        for c in range(nch):
            bi = c % 2
            if c + 1 < nch:
                if owrite is not None:
                    owrite.wait()
                    owrite = None
                nxt = issue(c + 1, 1 - bi)
            for cp in pend:
                cp.wait()
            if owrite is not None:
                owrite.wait()
                owrite = None
            w0all = w0v[bi] * SCALE
            w1all = w1v[bi] * SCALE

            def body_l(l, carry):
                a0 = _splat16(w0all, l)
                a1 = _splat16(w1all, l)
                for kk in range(H // L):
                    sl = pl.ds(kk * L, L)
                    shv[bi, l, sl] = (shv[bi, l, sl]
                                      + a0 * y0[bi, l, sl] + a1 * y1[bi, l, sl])
                return carry

            lax.fori_loop(0, L, body_l, 0)
            tb = wid * TW + c * L
            owrite = pltpu.async_copy(shv.at[bi], o_hbm.at[pl.ds(tb, L)],
                                      sems[bi][4])
            if c + 1 < nch:
                pend = nxt
        owrite.wait()

    return k(y, pos0, pos1, w0, w1, sh)


def kernel(hidden_states, gate_w, w_gate_up, w_down, shared_gate_up, shared_down):
    x = hidden_states.reshape(T, H)
    shared_out, e0, e1, w0, w1, counts = _shared_and_router(
        x, shared_gate_up, shared_down, gate_w)
    e0 = e0.reshape(T)
    e1 = e1.reshape(T)
    counts = counts.reshape(NW, E)

    xs, pos0, pos1, meta = _dispatch_sc(x, e0, e1, counts)
    y = _grouped_matmul(xs, w_gate_up, w_down, meta)
    out = _combine_sc(y, pos0, pos1, w0.reshape(T), w1.reshape(T), shared_out)
    return out.reshape(T, H)


# speedup vs baseline: 1.1165x; 1.0103x over previous
import functools

import jax
import jax.numpy as jnp
from jax import lax
from jax.experimental import pallas as pl
from jax.experimental.pallas import tpu as pltpu
from jax.experimental.pallas import tpu_sc as plsc

T = 2048
H = 1024
E = 16
I = 512
TOPK = 2
NG = 4
TG = 2
ISH = 1024
SCALE = 1.0

TBLK = 256
B = 64
NPAD = T * TOPK + E * B
NBLK = NPAD // B
MLEN = 96

NC = 2
NS = 16
NW = NC * NS
TW = T // NW
L = 16


def _shared_body(x_ref, sgu_ref, sd_ref, so_ref):
    x = x_ref[...]
    h = jnp.dot(x, sgu_ref[...], preferred_element_type=jnp.float32)
    a = h[:, :ISH]
    b = h[:, ISH:]
    g = a * jax.nn.sigmoid(a) * b
    so_ref[...] = jnp.dot(g, sd_ref[...], preferred_element_type=jnp.float32)


def _shared_mlp(x, sgu, sd):
    return pl.pallas_call(
        _shared_body,
        grid=(T // TBLK,),
        in_specs=[
            pl.BlockSpec((TBLK, H), lambda i: (i, 0)),
            pl.BlockSpec((H, 2 * ISH), lambda i: (0, 0)),
            pl.BlockSpec((ISH, H), lambda i: (0, 0)),
        ],
        out_specs=pl.BlockSpec((TBLK, H), lambda i: (i, 0)),
        out_shape=jax.ShapeDtypeStruct((T, H), jnp.float32),
    )(x, sgu, sd)


def _router_body(x_ref, gw_ref,
                 e0_ref, e1_ref, w0_ref, w1_ref, cnt_ref):
    x = x_ref[...]
    logits = jax.lax.dot_general(
        x, gw_ref[...], (((1,), (1,)), ((), ())),
        preferred_element_type=jnp.float32)
    m = jnp.max(logits, axis=1, keepdims=True)
    ex = jnp.exp(logits - m)
    sc = ex / jnp.sum(ex, axis=1, keepdims=True)

    gs = [jnp.max(sc[:, 4 * k:4 * k + 4], axis=1, keepdims=True)
          for k in range(NG)]
    col = jax.lax.broadcasted_iota(jnp.int32, (TBLK, E), 1)
    colg = col // (E // NG)
    masked = sc
    for j in range(NG):
        r = jnp.zeros((TBLK, 1), jnp.int32)
        for k in range(NG):
            gt = gs[k] > gs[j]
            tie = (gs[k] == gs[j]) & (k < j)
            r = r + jnp.where(gt | tie, 1, 0)
        keep = r < TG
        masked = jnp.where((colg == j) & jnp.logical_not(keep), 0.0, masked)

    m0 = jnp.max(masked, axis=1, keepdims=True)
    i0 = jnp.min(jnp.where(masked == m0, col, E), axis=1, keepdims=True)
    masked2 = jnp.where(col == i0, -1.0, masked)
    m1 = jnp.max(masked2, axis=1, keepdims=True)
    i1 = jnp.min(jnp.where(masked2 == m1, col, E), axis=1, keepdims=True)
    s = m0 + m1 + 1e-20
    e0_ref[...] = i0
    e1_ref[...] = i1
    w0_ref[...] = m0 / s
    w1_ref[...] = m1 / s

    for sub in range(TBLK // TW):
        s0 = i0[sub * TW:(sub + 1) * TW]
        s1 = i1[sub * TW:(sub + 1) * TW]
        cole = jax.lax.broadcasted_iota(jnp.int32, (TW, E), 1)
        cnt = (jnp.sum(jnp.where(s0 == cole, 1, 0), axis=0, keepdims=True)
               + jnp.sum(jnp.where(s1 == cole, 1, 0), axis=0, keepdims=True))
        cnt_ref[sub] = cnt


def _router(x, gw):
    grid = (T // TBLK,)
    return pl.pallas_call(
        _router_body,
        grid=grid,
        in_specs=[
            pl.BlockSpec((TBLK, H), lambda i: (i, 0)),
            pl.BlockSpec((E, H), lambda i: (0, 0)),
        ],
        out_specs=[
            pl.BlockSpec((TBLK, 1), lambda i: (i, 0)),
            pl.BlockSpec((TBLK, 1), lambda i: (i, 0)),
            pl.BlockSpec((TBLK, 1), lambda i: (i, 0)),
            pl.BlockSpec((TBLK, 1), lambda i: (i, 0)),
            pl.BlockSpec((TBLK // TW, 1, E), lambda i: (i, 0, 0)),
        ],
        out_shape=[
            jax.ShapeDtypeStruct((T, 1), jnp.int32),
            jax.ShapeDtypeStruct((T, 1), jnp.int32),
            jax.ShapeDtypeStruct((T, 1), jnp.float32),
            jax.ShapeDtypeStruct((T, 1), jnp.float32),
            jax.ShapeDtypeStruct((NW, 1, E), jnp.int32),
        ],
    )(x, gw)


_GDN = jax.lax.GatherDimensionNumbers(
    offset_dims=(), collapsed_slice_dims=(0,), start_index_map=(0,))


def _gather16(v, idx):
    return jax.lax.gather(
        v, idx.reshape(L, 1), _GDN, (1,),
        mode=jax.lax.GatherScatterMode.PROMISE_IN_BOUNDS)


def _splat16(v, i):
    return _gather16(v, jnp.zeros((L,), jnp.int32) + i)


def _eq16(a, b):
    d = a ^ b
    return 1 - (((d | (0 - d)) >> 31) & 1)


def _ge16(a, b):
    return 1 - (((a - b) >> 31) & 1)


def _cumsum16(x):
    lanes = jax.lax.iota(jnp.int32, L)
    y = x
    for s in (1, 2, 4, 8):
        g = _gather16(y, jnp.maximum(lanes - s, 0))
        m = ((s - 1 - lanes) >> 31) & 1
        y = y + g * m
    return y


def _dispatch_sc(x, e0, e1, counts):
    mesh = plsc.VectorSubcoreMesh(core_axis_name="c", subcore_axis_name="s")

    @functools.partial(
        pl.kernel,
        out_type=[
            jax.ShapeDtypeStruct((NPAD, H), jnp.float32),
            jax.ShapeDtypeStruct((T,), jnp.int32),
            jax.ShapeDtypeStruct((T,), jnp.int32),
            jax.ShapeDtypeStruct((MLEN,), jnp.int32),
        ],
        mesh=mesh,
        scratch_types=[
            pltpu.VMEM((TW,), jnp.int32),
            pltpu.VMEM((TW,), jnp.int32),
            pltpu.VMEM((TW,), jnp.int32),
            pltpu.VMEM((TW,), jnp.int32),
            pltpu.VMEM((TW, H), jnp.float32),
            pltpu.VMEM((NW, E), jnp.int32),
            pltpu.VMEM((MLEN,), jnp.int32),
            pltpu.SemaphoreType.DMA,
            pltpu.SemaphoreType.DMA,
        ],
    )
    def k(x_hbm, e0_hbm, e1_hbm, c_hbm, xs_hbm, p0_hbm, p1_hbm, meta_hbm,
          e0v, e1v, p0v, p1v, xrows, cnts, metav, semx, sems):
        wid = lax.axis_index("s") * NC + lax.axis_index("c")
        tb = wid * TW
        cpx = pltpu.async_copy(x_hbm.at[pl.ds(tb, TW)], xrows, semx)
        pltpu.sync_copy(e0_hbm.at[pl.ds(tb, TW)], e0v)
        pltpu.sync_copy(e1_hbm.at[pl.ds(tb, TW)], e1v)
        pltpu.sync_copy(c_hbm, cnts)

        lanes = jax.lax.iota(jnp.int32, L)
        totals = jnp.zeros((L,), jnp.int32)
        baserel = jnp.zeros((L,), jnp.int32)
        for w2 in range(NW):
            row = cnts[w2]
            totals = totals + row
            baserel = baserel + row * (((w2 - wid) >> 31) & 1)

        nblk = (totals + (B - 1)) >> 6
        cblk = _cumsum16(nblk)
        offb = cblk - nblk
        base = (offb << 6) + baserel

        for c in range(TW // L):
            ev0 = e0v[pl.ds(c * L, L)]
            ev1 = e1v[pl.ds(c * L, L)]
            p0a = jnp.zeros((L,), jnp.int32)
            p1a = jnp.zeros((L,), jnp.int32)
            for e in range(E):
                b_e = _splat16(base, e)
                m0i = _eq16(ev0, jnp.full((L,), e, jnp.int32))
                cs0 = _cumsum16(m0i)
                p0a = p0a + (b_e + cs0 - 1 - p0a) * m0i
                n0 = _splat16(cs0, L - 1)
                m1i = _eq16(ev1, jnp.full((L,), e, jnp.int32))
                cs1 = _cumsum16(m1i)
                p1a = p1a + (b_e + n0 + cs1 - 1 - p1a) * m1i
                n1 = _splat16(cs1, L - 1)
                base = base + _eq16(lanes, jnp.full((L,), e, jnp.int32)) * (n0 + n1)
            p0v[pl.ds(c * L, L)] = p0a
            p1v[pl.ds(c * L, L)] = p1a

        pltpu.sync_copy(p0v, p0_hbm.at[pl.ds(tb, TW)])
        pltpu.sync_copy(p1v, p1_hbm.at[pl.ds(tb, TW)])
        cpx.wait()
        cpa = pltpu.async_copy(xrows, xs_hbm.at[p0v], sems)
        cpb = pltpu.async_copy(xrows, xs_hbm.at[p1v], semx)
        cpa.wait()
        cpb.wait()

        @pl.when(wid == 0)
        def _():
            used = _splat16(cblk, L - 1)
            for k6 in range(MLEN // L):
                biota = lanes + L * k6
                acc = jnp.full((L,), -1, jnp.int32)
                for e in range(E):
                    ob = _splat16(offb, e)
                    acc = acc + _ge16(biota, ob)
                acc = jnp.minimum(acc, E - 1)
                if k6 == NBLK // L:
                    acc = acc + (used - acc) * _eq16(
                        lanes, jnp.full((L,), NBLK % L, jnp.int32))
                metav[pl.ds(k6 * L, L)] = acc
            pltpu.sync_copy(metav, meta_hbm)

    return k(x, e0, e1, counts)


def _gmm_body(meta_ref, x_ref, wgu_ref, wd_ref, y_ref):
    i = pl.program_id(0)

    @pl.when(i < meta_ref[NBLK])
    def _():
        h = jnp.dot(x_ref[...], wgu_ref[0], preferred_element_type=jnp.float32)
        a = h[:, :I]
        b = h[:, I:]
        g = a * jax.nn.sigmoid(a) * b
        y_ref[...] = jnp.dot(g, wd_ref[0], preferred_element_type=jnp.float32)


def _grouped_matmul(x_sorted, wgu, wd, meta):
    grid_spec = pltpu.PrefetchScalarGridSpec(
        num_scalar_prefetch=1,
        grid=(NBLK,),
        in_specs=[
            pl.BlockSpec((B, H),
                         lambda i, m: (jnp.where(i < m[NBLK], i, NBLK - 1), 0)),
            pl.BlockSpec((1, H, 2 * I), lambda i, m: (m[i], 0, 0)),
            pl.BlockSpec((1, I, H), lambda i, m: (m[i], 0, 0)),
        ],
        out_specs=pl.BlockSpec(
            (B, H), lambda i, m: (jnp.where(i < m[NBLK], i, NBLK - 1), 0)),
    )
    return pl.pallas_call(
        _gmm_body,
        grid_spec=grid_spec,
        out_shape=jax.ShapeDtypeStruct((NPAD, H), jnp.float32),
    )(meta, x_sorted, wgu, wd)


def _combine_sc(y, pos0, pos1, w0, w1, sh):
    mesh = plsc.VectorSubcoreMesh(core_axis_name="c", subcore_axis_name="s")

    @functools.partial(
        pl.kernel,
        out_type=jax.ShapeDtypeStruct((T, H), jnp.float32),
        mesh=mesh,
        scratch_types=[
            pltpu.VMEM((2, L), jnp.int32),
            pltpu.VMEM((2, L), jnp.int32),
            pltpu.VMEM((2, L, H), jnp.float32),
            pltpu.VMEM((2, L, H), jnp.float32),
            pltpu.VMEM((2, L, H), jnp.float32),
            pltpu.VMEM((2, L), jnp.float32),
            pltpu.VMEM((2, L), jnp.float32),
            pltpu.SemaphoreType.DMA,
            pltpu.SemaphoreType.DMA,
            pltpu.SemaphoreType.DMA,
            pltpu.SemaphoreType.DMA,
            pltpu.SemaphoreType.DMA,
            pltpu.SemaphoreType.DMA,
            pltpu.SemaphoreType.DMA,
            pltpu.SemaphoreType.DMA,
            pltpu.SemaphoreType.DMA,
            pltpu.SemaphoreType.DMA,
        ],
    )
    def k(y_hbm, p0_hbm, p1_hbm, w0_hbm, w1_hbm, sh_hbm, o_hbm,
          idx0, idx1, y0, y1, shv, w0v, w1v,
          sy0a, sy1a, ssha, swa, soa, sy0b, sy1b, sshb, swb, sob):
        wid = lax.axis_index("s") * NC + lax.axis_index("c")
        nch = TW // L
        sems = [(sy0a, sy1a, ssha, swa, soa), (sy0b, sy1b, sshb, swb, sob)]

        def issue(c, bi):
            tb = wid * TW + c * L
            sy0, sy1, ssh, sw, _ = sems[bi]
            pltpu.sync_copy(p0_hbm.at[pl.ds(tb, L)], idx0.at[bi])
            pltpu.sync_copy(p1_hbm.at[pl.ds(tb, L)], idx1.at[bi])
            cps = (pltpu.async_copy(y_hbm.at[idx0.at[bi]], y0.at[bi], sy0),
                   pltpu.async_copy(y_hbm.at[idx1.at[bi]], y1.at[bi], sy1),
                   pltpu.async_copy(sh_hbm.at[pl.ds(tb, L)], shv.at[bi], ssh),
                   pltpu.async_copy(w0_hbm.at[pl.ds(tb, L)], w0v.at[bi], sw),
                   pltpu.async_copy(w1_hbm.at[pl.ds(tb, L)], w1v.at[bi], sw))
            return cps

        pend = issue(0, 0)
        owrite = None
        for c in range(nch):
            bi = c % 2
            if c + 1 < nch:
                if owrite is not None:
                    owrite.wait()
                    owrite = None
                nxt = issue(c + 1, 1 - bi)
            for cp in pend:
                cp.wait()
            if owrite is not None:
                owrite.wait()
                owrite = None
            w0all = w0v[bi] * SCALE
            w1all = w1v[bi] * SCALE

            def body_l(l, carry):
                a0 = _splat16(w0all, l)
                a1 = _splat16(w1all, l)
                for kk in range(H // L):
                    sl = pl.ds(kk * L, L)
                    shv[bi, l, sl] = (shv[bi, l, sl]
                                      + a0 * y0[bi, l, sl] + a1 * y1[bi, l, sl])
                return carry

            lax.fori_loop(0, L, body_l, 0)
            tb = wid * TW + c * L
            owrite = pltpu.async_copy(shv.at[bi], o_hbm.at[pl.ds(tb, L)],
                                      sems[bi][4])
            if c + 1 < nch:
                pend = nxt
        owrite.wait()

    return k(y, pos0, pos1, w0, w1, sh)


def kernel(hidden_states, gate_w, w_gate_up, w_down, shared_gate_up, shared_down):
    x = hidden_states.reshape(T, H)
    e0, e1, w0, w1, counts = _router(x, gate_w)
    e0 = e0.reshape(T)
    e1 = e1.reshape(T)
    counts = counts.reshape(NW, E)

    xs, pos0, pos1, meta = _dispatch_sc(x, e0, e1, counts)
    shared_out = _shared_mlp(x, shared_gate_up, shared_down)
    y = _grouped_matmul(xs, w_gate_up, w_down, meta)
    out = _combine_sc(y, pos0, pos1, w0.reshape(T), w1.reshape(T), shared_out)
    return out.reshape(T, H)
